# initial kernel scaffold (unmeasured)
import jax
import jax.numpy as jnp
from jax import lax
from jax.experimental import pallas as pl
from jax.experimental.pallas import tpu as pltpu

N_DEV = 16
M_BLK = 512
N_OUT = 4096


def kernel(x, w_mat):
    m_glob, k_shard = x.shape
    assert m_glob == N_DEV * M_BLK and k_shard == M_BLK

    def body(x_ref, w_ref, out_ref, send_buf, comm_ref, send_sems, recv_sems):
        t = pl.program_id(0)
        my = lax.axis_index("i")

        @pl.when(t == 0)
        def _setup():
            barrier = pltpu.get_barrier_semaphore()
            for d in range(N_DEV):
                pl.semaphore_signal(
                    barrier,
                    inc=1,
                    device_id=(d,),
                    device_id_type=pl.DeviceIdType.MESH,
                )
            pl.semaphore_wait(barrier, N_DEV)

            for j in range(N_DEV):
                send_buf[j] = x_ref[pl.ds(j * M_BLK, M_BLK), :].astype(
                    jnp.bfloat16
                )

                @pl.when(my != j)
                def _send(j=j):
                    rdma = pltpu.make_async_remote_copy(
                        src_ref=send_buf.at[j],
                        dst_ref=comm_ref.at[my],
                        send_sem=send_sems.at[j],
                        recv_sem=recv_sems.at[my],
                        device_id=(j,),
                        device_id_type=pl.DeviceIdType.MESH,
                    )
                    rdma.start()

                @pl.when(my == j)
                def _local(j=j):
                    comm_ref[j] = send_buf[j]

        @pl.when(t != my)
        def _wait():
            recv = pltpu.make_async_remote_copy(
                src_ref=comm_ref.at[t],
                dst_ref=comm_ref.at[t],
                send_sem=send_sems.at[t],
                recv_sem=recv_sems.at[t],
                device_id=(0,),
                device_id_type=pl.DeviceIdType.MESH,
            )
            recv.wait_recv()

        contrib = jnp.dot(
            comm_ref[t],
            w_ref[...].astype(jnp.bfloat16),
            preferred_element_type=jnp.float32,
        )

        @pl.when(t == 0)
        def _init():
            out_ref[...] = contrib

        @pl.when(jnp.logical_and(t > 0, t < N_DEV - 1))
        def _acc():
            out_ref[...] += contrib

        @pl.when(t == N_DEV - 1)
        def _fin():
            y = out_ref[...] + contrib
            c = 0.7978845608028654
            out_ref[...] = 0.5 * y * (1.0 + jnp.tanh(c * (y + 0.044715 * y * y * y)))

            for j in range(N_DEV):
                @pl.when(my != j)
                def _drain(j=j):
                    rdma = pltpu.make_async_remote_copy(
                        src_ref=send_buf.at[j],
                        dst_ref=comm_ref.at[my],
                        send_sem=send_sems.at[j],
                        recv_sem=recv_sems.at[my],
                        device_id=(j,),
                        device_id_type=pl.DeviceIdType.MESH,
                    )
                    rdma.wait_send()

    return pl.pallas_call(
        body,
        grid=(N_DEV,),
        out_shape=jax.ShapeDtypeStruct((M_BLK, N_OUT), jnp.float32),
        in_specs=[
            pl.BlockSpec((m_glob, k_shard), lambda t: (0, 0)),
            pl.BlockSpec((M_BLK, N_OUT), lambda t: (t, 0)),
        ],
        out_specs=pl.BlockSpec((M_BLK, N_OUT), lambda t: (0, 0)),
        scratch_shapes=[
            pltpu.VMEM((N_DEV, M_BLK, M_BLK), jnp.bfloat16),
            pltpu.VMEM((N_DEV, M_BLK, M_BLK), jnp.bfloat16),
            pltpu.SemaphoreType.DMA((N_DEV,)),
            pltpu.SemaphoreType.DMA((N_DEV,)),
        ],
        compiler_params=pltpu.CompilerParams(collective_id=0),
    )(x, w_mat)


# baseline (device time: 135988 ns/iter reference)
import jax
import jax.numpy as jnp
from jax import lax
from jax.experimental import pallas as pl
from jax.experimental.pallas import tpu as pltpu

N_DEV = 16
M_BLK = 512
N_OUT = 4096


def kernel(x, w_mat):
    m_glob, k_shard = x.shape
    assert m_glob == N_DEV * M_BLK and k_shard == M_BLK

    def body(x_ref, w_ref, out_ref, send_buf, comm_ref, send_sems, recv_sems):
        t = pl.program_id(0)
        my = lax.axis_index("i")

        @pl.when(t == 0)
        def _setup():
            barrier = pltpu.get_barrier_semaphore()
            for d in range(N_DEV):
                pl.semaphore_signal(
                    barrier,
                    inc=1,
                    device_id=(d,),
                    device_id_type=pl.DeviceIdType.MESH,
                )
            pl.semaphore_wait(barrier, N_DEV)

            for j in range(N_DEV):
                send_buf[j] = x_ref[pl.ds(j * M_BLK, M_BLK), :].astype(
                    jnp.bfloat16
                )

                @pl.when(my != j)
                def _send(j=j):
                    rdma = pltpu.make_async_remote_copy(
                        src_ref=send_buf.at[j],
                        dst_ref=comm_ref.at[my],
                        send_sem=send_sems.at[j],
                        recv_sem=recv_sems.at[my],
                        device_id=(j,),
                        device_id_type=pl.DeviceIdType.MESH,
                    )
                    rdma.start()

                @pl.when(my == j)
                def _local(j=j):
                    comm_ref[j] = send_buf[j]

        @pl.when(t != my)
        def _wait():
            recv = pltpu.make_async_remote_copy(
                src_ref=comm_ref.at[t],
                dst_ref=comm_ref.at[t],
                send_sem=send_sems.at[t],
                recv_sem=recv_sems.at[t],
                device_id=(0,),
                device_id_type=pl.DeviceIdType.MESH,
            )
            recv.wait_recv()

        contrib = jnp.dot(
            comm_ref[t],
            w_ref[...].astype(jnp.bfloat16),
            preferred_element_type=jnp.float32,
        )

        @pl.when(t == 0)
        def _init():
            out_ref[...] = contrib

        @pl.when(jnp.logical_and(t > 0, t < N_DEV - 1))
        def _acc():
            out_ref[...] += contrib

        @pl.when(t == N_DEV - 1)
        def _fin():
            y = out_ref[...] + contrib
            c = 0.7978845608028654
            out_ref[...] = 0.5 * y * (1.0 + jnp.tanh(c * (y + 0.044715 * y * y * y)))

            for j in range(N_DEV):
                @pl.when(my != j)
                def _drain(j=j):
                    rdma = pltpu.make_async_remote_copy(
                        src_ref=send_buf.at[j],
                        dst_ref=comm_ref.at[my],
                        send_sem=send_sems.at[j],
                        recv_sem=recv_sems.at[my],
                        device_id=(j,),
                        device_id_type=pl.DeviceIdType.MESH,
                    )
                    rdma.wait_send()

    return pl.pallas_call(
        body,
        grid=(N_DEV,),
        out_shape=jax.ShapeDtypeStruct((M_BLK, N_OUT), jnp.float32),
        in_specs=[
            pl.BlockSpec((m_glob, k_shard), lambda t: (0, 0)),
            pl.BlockSpec((M_BLK, N_OUT), lambda t: (t, 0)),
        ],
        out_specs=pl.BlockSpec((M_BLK, N_OUT), lambda t: (0, 0)),
        scratch_shapes=[
            pltpu.VMEM((N_DEV, M_BLK, M_BLK), jnp.bfloat16),
            pltpu.VMEM((N_DEV, M_BLK, M_BLK), jnp.bfloat16),
            pltpu.SemaphoreType.DMA((N_DEV,)),
            pltpu.SemaphoreType.DMA((N_DEV,)),
        ],
        compiler_params=pltpu.CompilerParams(
            collective_id=0, vmem_limit_bytes=100 * 1024 * 1024
        ),
    )(x, w_mat)
